# Initial kernel scaffold; baseline (speedup 1.0000x reference)
import jax
import jax.numpy as jnp
from jax.experimental import pallas as pl


def kernel(elm_input_ids, signal_id_indices, signal_feats, embed_table, W_enc, W_lm):
    raise NotImplementedError("write your pallas kernel here")


# speedup vs baseline: 1.0274x; 1.0274x over previous
import functools

import jax
import jax.numpy as jnp
from jax.experimental import pallas as pl
from jax.experimental.pallas import tpu as pltpu
from jax.experimental.pallas import tpu_sc as plsc

_B, _S, _D, _V, _DENC = 4, 8192, 1024, 32000, 512
_BLK = 256
_NBLK = _V // _BLK
_VPAD = (_NBLK + 1) * _BLK
_NIDS = _B * _S
_WIN = 16


def _proj_body(sf_ref, wenc_ref, tab_ref, wlm_ref, out_ref):
    i = pl.program_id(0)
    wlm = wlm_ref[...].astype(jnp.bfloat16)

    @pl.when(i < _NBLK)
    def _():
        out_ref[...] = jnp.dot(
            tab_ref[...].astype(jnp.bfloat16), wlm,
            preferred_element_type=jnp.float32)

    @pl.when(i == _NBLK)
    def _():
        enc = jnp.dot(
            sf_ref[...].astype(jnp.bfloat16),
            wenc_ref[...].astype(jnp.bfloat16),
            preferred_element_type=jnp.float32)
        encp = jnp.dot(
            enc.astype(jnp.bfloat16), wlm,
            preferred_element_type=jnp.float32)
        out_ref[...] = jnp.pad(encp, ((0, _BLK - _B), (0, 0)))


def _project_table(signal_feats, W_enc, embed_table, W_lm):
    return pl.pallas_call(
        _proj_body,
        grid=(_NBLK + 1,),
        in_specs=[
            pl.BlockSpec((_B, _DENC), lambda i: (0, 0)),
            pl.BlockSpec((_DENC, _D), lambda i: (0, 0)),
            pl.BlockSpec((_BLK, _D), lambda i: (jnp.minimum(i, _NBLK - 1), 0)),
            pl.BlockSpec((_D, _D), lambda i: (0, 0)),
        ],
        out_specs=pl.BlockSpec((_BLK, _D), lambda i: (i, 0)),
        out_shape=jax.ShapeDtypeStruct((_VPAD, _D), jnp.float32),
    )(signal_feats, W_enc, embed_table, W_lm)


_NW = 32
_BPW = _NIDS // _NW
_CH = 32
_NCH = _BPW // _CH


@functools.cache
def _make_gather():
    from jax import lax

    @functools.partial(
        pl.kernel,
        out_type=jax.ShapeDtypeStruct((_NIDS, _D), jnp.float32),
        mesh=plsc.VectorSubcoreMesh(core_axis_name="c", subcore_axis_name="s"),
        scratch_types=[
            pltpu.VMEM((_BPW,), jnp.int32),
            pltpu.VMEM((_CH, _D), jnp.float32),
            pltpu.VMEM((_CH, _D), jnp.float32),
            pltpu.SemaphoreType.DMA,
            pltpu.SemaphoreType.DMA,
        ],
    )
    def _gather_k(t_hbm, i_hbm, o_hbm, idx_v, r0, r1, sem0, sem1):
        wid = lax.axis_index("s") * 2 + lax.axis_index("c")
        base = wid * _BPW
        pltpu.sync_copy(i_hbm.at[pl.ds(base, _BPW)], idx_v)

        def start(chunk, buf, sem):
            pltpu.make_async_copy(
                t_hbm.at[idx_v.at[pl.ds(chunk * _CH, _CH)]], buf, sem).start()

        def wait_and_store(chunk, buf, sem):
            pltpu.make_async_copy(
                t_hbm.at[idx_v.at[pl.ds(chunk * _CH, _CH)]], buf, sem).wait()
            pltpu.sync_copy(buf, o_hbm.at[pl.ds(base + chunk * _CH, _CH)])

        start(0, r0, sem0)

        @pl.loop(0, _NCH, step=2)
        def _(g):
            start(g + 1, r1, sem1)
            wait_and_store(g, r0, sem0)

            @pl.when(g + 2 < _NCH)
            def _():
                start(g + 2, r0, sem0)

            wait_and_store(g + 1, r1, sem1)

    return _gather_k


def kernel(elm_input_ids, signal_id_indices, signal_feats, embed_table,
           W_enc, W_lm):
    bidx = jnp.arange(_B, dtype=jnp.int32)
    ids = elm_input_ids.at[bidx, signal_id_indices].set(_V + bidx)
    t_ext = _project_table(signal_feats, W_enc, embed_table, W_lm)
    out = _make_gather()(t_ext, ids.reshape(_NIDS))
    return out.reshape(_B, _S, _D)


# speedup vs baseline: 1.2587x; 1.2251x over previous
import functools

import jax
import jax.numpy as jnp
from jax import lax
from jax.experimental import pallas as pl
from jax.experimental.pallas import tpu as pltpu
from jax.experimental.pallas import tpu_sc as plsc

_B, _S, _D, _V, _DENC = 4, 8192, 1024, 32000, 512
_NIDS = _B * _S
_NCHUNK = 4
_CROWS = _NIDS // _NCHUNK
_BLK = 512
_NW = 32
_BPW = _CROWS // _NW
_CH = 32
_NCH = _BPW // _CH


@functools.cache
def _make_gather():
    @functools.partial(
        pl.kernel,
        out_type=jax.ShapeDtypeStruct((_CROWS, _D), jnp.float32),
        mesh=plsc.VectorSubcoreMesh(core_axis_name="c", subcore_axis_name="s"),
        scratch_types=[
            pltpu.VMEM((_BPW,), jnp.int32),
            pltpu.VMEM((_CH, _D), jnp.float32),
            pltpu.VMEM((_CH, _D), jnp.float32),
            pltpu.SemaphoreType.DMA,
            pltpu.SemaphoreType.DMA,
        ],
    )
    def _gather_k(t_hbm, i_hbm, o_hbm, idx_v, r0, r1, sem0, sem1):
        wid = lax.axis_index("s") * 2 + lax.axis_index("c")
        base = wid * _BPW
        pltpu.sync_copy(i_hbm.at[pl.ds(base, _BPW)], idx_v)

        def start(chunk, buf, sem):
            pltpu.make_async_copy(
                t_hbm.at[idx_v.at[pl.ds(chunk * _CH, _CH)]], buf, sem).start()

        def wait_and_store(chunk, buf, sem):
            pltpu.make_async_copy(
                t_hbm.at[idx_v.at[pl.ds(chunk * _CH, _CH)]], buf, sem).wait()
            pltpu.sync_copy(buf, o_hbm.at[pl.ds(base + chunk * _CH, _CH)])

        start(0, r0, sem0)

        @pl.loop(0, _NCH, step=2)
        def _(g):
            start(g + 1, r1, sem1)
            wait_and_store(g, r0, sem0)

            @pl.when(g + 2 < _NCH)
            def _():
                start(g + 2, r0, sem0)

            wait_and_store(g + 1, r1, sem1)

    return _gather_k


def _mm_chunk(k, e_k, pos, sf_bf, wenc_bf, wlm_bf, out_prev):
    def body(*refs):
        pos_ref, sf_ref, wenc_ref, wlm_ref, e_ref = refs[:5]
        o_ref = refs[-1]
        i = pl.program_id(0)
        r0 = k * _CROWS + i * _BLK
        wlm = wlm_ref[...]
        o_ref[...] = jnp.dot(e_ref[...].astype(jnp.bfloat16), wlm,
                             preferred_element_type=jnp.float32)
        enc = jnp.dot(sf_ref[...], wenc_ref[...],
                      preferred_element_type=jnp.float32)
        encp = jnp.dot(enc.astype(jnp.bfloat16), wlm,
                       preferred_element_type=jnp.float32)
        for b in range(_B):
            p = pos_ref[b]

            @pl.when((p >= r0) & (p < r0 + _BLK))
            def _():
                o_ref[pl.ds(p - r0, 1), :] = encp[b:b + 1, :]

    nsteps = _CROWS // _BLK
    base_blk = k * nsteps
    in_specs = [
        pl.BlockSpec(memory_space=pltpu.MemorySpace.SMEM),
        pl.BlockSpec((_B, _DENC), lambda i: (0, 0)),
        pl.BlockSpec((_DENC, _D), lambda i: (0, 0)),
        pl.BlockSpec((_D, _D), lambda i: (0, 0)),
        pl.BlockSpec((_BLK, _D), lambda i: (i, 0)),
    ]
    args = [pos, sf_bf, wenc_bf, wlm_bf, e_k]
    aliases = {}
    if out_prev is not None:
        in_specs.append(pl.BlockSpec(memory_space=pl.ANY))
        args.append(out_prev)
        aliases = {5: 0}
    return pl.pallas_call(
        body,
        grid=(nsteps,),
        in_specs=in_specs,
        out_specs=pl.BlockSpec((_BLK, _D), lambda i: (base_blk + i, 0)),
        out_shape=jax.ShapeDtypeStruct((_NIDS, _D), jnp.float32),
        input_output_aliases=aliases,
    )(*args)


def kernel(elm_input_ids, signal_id_indices, signal_feats, embed_table,
           W_enc, W_lm):
    ids = elm_input_ids.reshape(_NIDS)
    pos = (jnp.arange(_B, dtype=jnp.int32) * _S
           + signal_id_indices.astype(jnp.int32))
    sf_bf = signal_feats.astype(jnp.bfloat16)
    wenc_bf = W_enc.astype(jnp.bfloat16)
    wlm_bf = W_lm.astype(jnp.bfloat16)

    gather = _make_gather()
    out = None
    for k in range(_NCHUNK):
        e_k = gather(embed_table, lax.dynamic_slice(ids, (k * _CROWS,),
                                                    (_CROWS,)))
        out = _mm_chunk(k, e_k, pos, sf_bf, wenc_bf, wlm_bf, out)
    return out.reshape(_B, _S, _D)
